# baseline (speedup 1.0000x reference)
import functools

import jax
import jax.numpy as jnp
from jax import lax
from jax.experimental import pallas as pl
from jax.experimental.pallas import tpu as pltpu
from jax.experimental.pallas import tpu_sc as plsc

N = 10000
E = 320000
NFEAT = 128
NHID = 64
NCLS = 64
KSTEPS = 10

FH = 32
NTILES = 16
NACC = 10112
NPT = NACC // NTILES
ECW = 256
EPT_ROWS = 80
EROWS = NTILES * EPT_ROWS
EPAD = EROWS * ECW
DEG_ROWS = EPAD // 128 // 32
TRASH = 10008

_MESH = plsc.VectorSubcoreMesh(core_axis_name="c", subcore_axis_name="s")


@functools.partial(
    pl.kernel,
    out_type=[
        jax.ShapeDtypeStruct((NACC, FH), jnp.float32),
        jax.ShapeDtypeStruct((NACC, FH), jnp.float32),
    ],
    mesh=_MESH,
    compiler_params=pltpu.CompilerParams(use_tc_tiling_on_sc=False),
    scratch_types=[
        pltpu.VMEM_SHARED((NACC, FH), jnp.float32),
        pltpu.VMEM((NPT, FH), jnp.float32),
        pltpu.VMEM((DEG_ROWS, 128), jnp.int32),
        pltpu.VMEM((128, FH), jnp.float32),
    ],
)
def _deg_kernel(dstr, zeros, ones, degl, degr, acc, zbuf, dstbuf, onesbuf):
    c = lax.axis_index("c")
    s = lax.axis_index("s")
    w = c * NTILES + s
    pltpu.sync_copy(zeros.at[pl.ds(s * NPT, NPT)], zbuf)
    pltpu.sync_copy(zbuf, acc.at[pl.ds(s * NPT, NPT)])
    pltpu.sync_copy(ones, onesbuf)
    pltpu.sync_copy(dstr.at[pl.ds(w * DEG_ROWS, DEG_ROWS)], dstbuf)
    plsc.subcore_barrier()

    def ej(j, carry):
        pltpu.sync_copy(onesbuf, acc.at[dstbuf.at[j]], add=True)
        return carry

    lax.fori_loop(0, DEG_ROWS, ej, 0)
    plsc.subcore_barrier()
    pltpu.sync_copy(acc.at[pl.ds(s * NPT, NPT)], zbuf)

    @pl.when(c == 0)
    def _():
        pltpu.sync_copy(zbuf, degl.at[pl.ds(s * NPT, NPT)])

    @pl.when(c == 1)
    def _():
        pltpu.sync_copy(zbuf, degr.at[pl.ds(s * NPT, NPT)])


def _mlp_body(x_ref, w1_ref, b1_ref, w2_ref, b2_ref, dl_ref, dr_ref,
              c9_ref, q0l_ref, q0r_ref, sd_ref):
    xb = x_ref[...]
    h1 = lax.dot_general(xb, w1_ref[...], (((1,), (1,)), ((), ())),
                         preferred_element_type=jnp.float32)
    h1 = jnp.maximum(h1 + b1_ref[...], 0.0)
    h0 = lax.dot_general(h1, w2_ref[...], (((1,), (1,)), ((), ())),
                         preferred_element_type=jnp.float32)
    h0 = h0 + b2_ref[...]
    deg = dl_ref[...] + dr_ref[...] + 1.0
    c9_ref[...] = 0.9 / deg
    dis = lax.rsqrt(deg)
    sd_ref[...] = deg * dis
    q0l_ref[...] = dis * h0[:, :FH]
    q0r_ref[...] = dis * h0[:, FH:]


def _run_mlp(x, W1, b1, W2, b2, degl, degr):
    blk = 1000
    grid = N // blk
    f32 = jnp.float32
    return pl.pallas_call(
        _mlp_body,
        grid=(grid,),
        in_specs=[
            pl.BlockSpec((blk, NFEAT), lambda i: (i, 0)),
            pl.BlockSpec((NHID, NFEAT), lambda i: (0, 0)),
            pl.BlockSpec((1, NHID), lambda i: (0, 0)),
            pl.BlockSpec((NCLS, NHID), lambda i: (0, 0)),
            pl.BlockSpec((1, NCLS), lambda i: (0, 0)),
            pl.BlockSpec((blk, FH), lambda i: (i, 0)),
            pl.BlockSpec((blk, FH), lambda i: (i, 0)),
        ],
        out_specs=[
            pl.BlockSpec((blk, FH), lambda i: (i, 0)),
            pl.BlockSpec((blk, FH), lambda i: (i, 0)),
            pl.BlockSpec((blk, FH), lambda i: (i, 0)),
            pl.BlockSpec((blk, FH), lambda i: (i, 0)),
        ],
        out_shape=[
            jax.ShapeDtypeStruct((N, FH), f32),
            jax.ShapeDtypeStruct((N, FH), f32),
            jax.ShapeDtypeStruct((N, FH), f32),
            jax.ShapeDtypeStruct((N, FH), f32),
        ],
    )(x, W1, b1.reshape(1, NHID), W2, b2.reshape(1, NCLS), degl, degr)


@functools.partial(
    pl.kernel,
    out_type=[
        jax.ShapeDtypeStruct((NACC, FH), jnp.float32),
        jax.ShapeDtypeStruct((NACC, FH), jnp.float32),
    ],
    mesh=_MESH,
    compiler_params=pltpu.CompilerParams(use_tc_tiling_on_sc=False),
    scratch_types=[
        pltpu.VMEM_SHARED((NACC, FH), jnp.float32),
        pltpu.VMEM((EPT_ROWS, ECW), jnp.int32),
        pltpu.VMEM((EPT_ROWS, ECW), jnp.int32),
        pltpu.VMEM((NPT, FH), jnp.float32),
        pltpu.VMEM((NPT, FH), jnp.float32),
        pltpu.VMEM((NPT, FH), jnp.float32),
        pltpu.VMEM((ECW, FH), jnp.float32),
        pltpu.SemaphoreType.DMA,
    ],
)
def _prop_kernel(srcr, dstr, c9, q0l, q0r, qlo, qro,
                 acc, srcbuf, dstbuf, qnew, cbuf, rbuf,
                 rw0, sga):
    c = lax.axis_index("c")
    s = lax.axis_index("s")
    r0 = s * NPT
    pltpu.sync_copy(srcr.at[pl.ds(s * EPT_ROWS, EPT_ROWS)], srcbuf)
    pltpu.sync_copy(dstr.at[pl.ds(s * EPT_ROWS, EPT_ROWS)], dstbuf)
    pltpu.sync_copy(c9.at[pl.ds(r0, NPT)], cbuf)

    @pl.when(c == 0)
    def _():
        pltpu.sync_copy(q0l.at[pl.ds(r0, NPT)], qnew)

    @pl.when(c == 1)
    def _():
        pltpu.sync_copy(q0r.at[pl.ds(r0, NPT)], qnew)

    def rb(i, carry):
        rbuf[i, pl.ds(0, 16)] = qnew[i, pl.ds(0, 16)] * 0.1
        rbuf[i, pl.ds(16, 16)] = qnew[i, pl.ds(16, 16)] * 0.1
        return carry

    lax.fori_loop(0, NPT, rb, 0)

    def run(qout):
        def kbody(k, carry):
            pltpu.sync_copy(qnew, qout.at[pl.ds(r0, NPT)])
            pltpu.sync_copy(qnew, acc.at[pl.ds(r0, NPT)])
            plsc.subcore_barrier()

            def ej(j, icarry):
                pltpu.async_copy(qout.at[srcbuf.at[j]], rw0, sga).wait()
                pltpu.sync_copy(rw0, acc.at[dstbuf.at[j]], add=True)
                return icarry

            lax.fori_loop(0, EPT_ROWS, ej, 0)
            plsc.subcore_barrier()
            pltpu.sync_copy(acc.at[pl.ds(r0, NPT)], qnew)

            def cb(i, icarry):
                qnew[i, pl.ds(0, 16)] = (cbuf[i, pl.ds(0, 16)]
                                         * qnew[i, pl.ds(0, 16)]
                                         + rbuf[i, pl.ds(0, 16)])
                qnew[i, pl.ds(16, 16)] = (cbuf[i, pl.ds(16, 16)]
                                          * qnew[i, pl.ds(16, 16)]
                                          + rbuf[i, pl.ds(16, 16)])
                return icarry

            lax.fori_loop(0, NPT, cb, 0)
            return carry

        lax.fori_loop(0, KSTEPS, kbody, 0)
        pltpu.sync_copy(qnew, qout.at[pl.ds(r0, NPT)])

    @pl.when(c == 0)
    def _():
        run(qlo)

    @pl.when(c == 1)
    def _():
        run(qro)


def _ls_body(q_ref, sd_ref, o_ref):
    sd = sd_ref[...]
    h = q_ref[...] * jnp.concatenate([sd, sd], axis=1)
    m = jnp.max(h, axis=1, keepdims=True)
    e = jnp.exp(h - m)
    o_ref[...] = (h - m) - jnp.log(jnp.sum(e, axis=1, keepdims=True))


def _run_log_softmax(q, sdeg):
    blk = 1000
    return pl.pallas_call(
        _ls_body,
        grid=(N // blk,),
        in_specs=[
            pl.BlockSpec((blk, NCLS), lambda i: (i, 0)),
            pl.BlockSpec((blk, FH), lambda i: (i, 0)),
        ],
        out_specs=pl.BlockSpec((blk, NCLS), lambda i: (i, 0)),
        out_shape=jax.ShapeDtypeStruct((N, NCLS), jnp.float32),
    )(q, sdeg)


def kernel(x, edge_index, W1, b1, W2, b2):
    src = edge_index[0].astype(jnp.int32)
    dst = edge_index[1].astype(jnp.int32)
    npad = EPAD - E
    src_p = jnp.concatenate([src, jnp.zeros((npad,), jnp.int32)]).reshape(EROWS, ECW)
    dst_flat = jnp.concatenate([dst, jnp.full((npad,), TRASH, jnp.int32)])
    dst_p = dst_flat.reshape(EROWS, ECW)
    dst_p128 = dst_flat.reshape(EPAD // 128, 128)
    zeros_a = jnp.zeros((NACC, FH), jnp.float32)
    ones_a = jnp.ones((128, FH), jnp.float32)

    degl, degr = _deg_kernel(dst_p128, zeros_a, ones_a)
    c9, q0l, q0r, sdeg = _run_mlp(x, W1, b1, W2, b2,
                                  degl[:N], degr[:N])
    pad = ((0, NACC - N), (0, 0))
    ql, qr = _prop_kernel(src_p, dst_p, jnp.pad(c9, pad),
                          jnp.pad(q0l, pad), jnp.pad(q0r, pad))
    q = jnp.concatenate([ql[:N], qr[:N]], axis=1)
    return _run_log_softmax(q, sdeg)


# speedup vs baseline: 1.0995x; 1.0995x over previous
import functools

import jax
import jax.numpy as jnp
from jax import lax
from jax.experimental import pallas as pl
from jax.experimental.pallas import tpu as pltpu
from jax.experimental.pallas import tpu_sc as plsc

N = 10000
E = 320000
NFEAT = 128
NHID = 64
NCLS = 64
KSTEPS = 10

FH = 32
NTILES = 16
NACC = 10240
NPT = NACC // NTILES
ECW = 256
EPT_ROWS = 80
EROWS = NTILES * EPT_ROWS
EPAD = EROWS * ECW
DEG_ROWS = EPAD // 128 // 32
TRASH = 10008

_MESH = plsc.VectorSubcoreMesh(core_axis_name="c", subcore_axis_name="s")


@functools.partial(
    pl.kernel,
    out_type=[
        jax.ShapeDtypeStruct((NACC, FH), jnp.float32),
        jax.ShapeDtypeStruct((NACC, FH), jnp.float32),
    ],
    mesh=_MESH,
    compiler_params=pltpu.CompilerParams(use_tc_tiling_on_sc=False),
    scratch_types=[
        pltpu.VMEM_SHARED((NACC, FH), jnp.float32),
        pltpu.VMEM((NPT, FH), jnp.float32),
        pltpu.VMEM((DEG_ROWS, 128), jnp.int32),
        pltpu.VMEM((128, FH), jnp.float32),
    ],
)
def _deg_kernel(dstr, zeros, ones, degl, degr, acc, zbuf, dstbuf, onesbuf):
    c = lax.axis_index("c")
    s = lax.axis_index("s")
    w = c * NTILES + s
    pltpu.sync_copy(zeros.at[pl.ds(s * NPT, NPT)], zbuf)
    pltpu.sync_copy(zbuf, acc.at[pl.ds(s * NPT, NPT)])
    pltpu.sync_copy(ones, onesbuf)
    pltpu.sync_copy(dstr.at[pl.ds(w * DEG_ROWS, DEG_ROWS)], dstbuf)
    plsc.subcore_barrier()

    def ej(j, carry):
        pltpu.sync_copy(onesbuf, acc.at[dstbuf.at[j]], add=True)
        return carry

    lax.fori_loop(0, DEG_ROWS, ej, 0)
    plsc.subcore_barrier()
    pltpu.sync_copy(acc.at[pl.ds(s * NPT, NPT)], zbuf)

    @pl.when(c == 0)
    def _():
        pltpu.sync_copy(zbuf, degl.at[pl.ds(s * NPT, NPT)])

    @pl.when(c == 1)
    def _():
        pltpu.sync_copy(zbuf, degr.at[pl.ds(s * NPT, NPT)])


def _mlp_body(x_ref, w1_ref, b1_ref, w2_ref, b2_ref, dl_ref, dr_ref,
              c9_ref, q0l_ref, q0r_ref, sd_ref):
    xb = x_ref[...]
    h1 = lax.dot_general(xb, w1_ref[...], (((1,), (1,)), ((), ())),
                         preferred_element_type=jnp.float32)
    h1 = jnp.maximum(h1 + b1_ref[...], 0.0)
    h0 = lax.dot_general(h1, w2_ref[...], (((1,), (1,)), ((), ())),
                         preferred_element_type=jnp.float32)
    h0 = h0 + b2_ref[...]
    deg = dl_ref[...] + dr_ref[...] + 1.0
    c9_ref[...] = 0.9 / deg
    dis = lax.rsqrt(deg)
    sd_ref[...] = deg * dis
    q0l_ref[...] = dis * h0[:, :FH]
    q0r_ref[...] = dis * h0[:, FH:]


def _run_mlp(x, W1, b1, W2, b2, degl, degr):
    blk = 1000
    grid = N // blk
    f32 = jnp.float32
    return pl.pallas_call(
        _mlp_body,
        grid=(grid,),
        in_specs=[
            pl.BlockSpec((blk, NFEAT), lambda i: (i, 0)),
            pl.BlockSpec((NHID, NFEAT), lambda i: (0, 0)),
            pl.BlockSpec((1, NHID), lambda i: (0, 0)),
            pl.BlockSpec((NCLS, NHID), lambda i: (0, 0)),
            pl.BlockSpec((1, NCLS), lambda i: (0, 0)),
            pl.BlockSpec((blk, FH), lambda i: (i, 0)),
            pl.BlockSpec((blk, FH), lambda i: (i, 0)),
        ],
        out_specs=[
            pl.BlockSpec((blk, FH), lambda i: (i, 0)),
            pl.BlockSpec((blk, FH), lambda i: (i, 0)),
            pl.BlockSpec((blk, FH), lambda i: (i, 0)),
            pl.BlockSpec((blk, FH), lambda i: (i, 0)),
        ],
        out_shape=[
            jax.ShapeDtypeStruct((N, FH), f32),
            jax.ShapeDtypeStruct((N, FH), f32),
            jax.ShapeDtypeStruct((N, FH), f32),
            jax.ShapeDtypeStruct((N, FH), f32),
        ],
    )(x, W1, b1.reshape(1, NHID), W2, b2.reshape(1, NCLS), degl, degr)


@functools.partial(
    pl.kernel,
    out_type=[
        jax.ShapeDtypeStruct((NACC, FH), jnp.float32),
        jax.ShapeDtypeStruct((NACC, FH), jnp.float32),
    ],
    mesh=_MESH,
    compiler_params=pltpu.CompilerParams(use_tc_tiling_on_sc=False),
    scratch_types=[
        pltpu.VMEM_SHARED((NACC, FH), jnp.float32),
        pltpu.VMEM((EPT_ROWS, ECW), jnp.int32),
        pltpu.VMEM((EPT_ROWS, ECW), jnp.int32),
        pltpu.VMEM((NPT, FH), jnp.float32),
        pltpu.VMEM((NPT, FH), jnp.float32),
        pltpu.VMEM((ECW, FH), jnp.float32),
        pltpu.VMEM((ECW, FH), jnp.float32),
        pltpu.VMEM((128, FH), jnp.float32),
        pltpu.SemaphoreType.DMA,
        pltpu.SemaphoreType.DMA,
        pltpu.SemaphoreType.DMA,
    ],
)
def _prop_kernel(srcr, dstr, c9, q0l, q0r, qlo, qro,
                 acc, srcbuf, dstbuf, qnew, cbuf,
                 rw0, rw1, rw2, sga, sgb, ssc):
    c = lax.axis_index("c")
    s = lax.axis_index("s")
    r0 = s * NPT
    pltpu.sync_copy(srcr.at[pl.ds(s * EPT_ROWS, EPT_ROWS)], srcbuf)
    pltpu.sync_copy(dstr.at[pl.ds(s * EPT_ROWS, EPT_ROWS)], dstbuf)
    pltpu.sync_copy(c9.at[pl.ds(r0, NPT)], cbuf)

    @pl.when(c == 0)
    def _():
        pltpu.sync_copy(q0l.at[pl.ds(r0, NPT)], qnew)

    @pl.when(c == 1)
    def _():
        pltpu.sync_copy(q0r.at[pl.ds(r0, NPT)], qnew)

    def run(qout, q0ref):
        def kbody(k, carry):
            pltpu.sync_copy(qnew, qout.at[pl.ds(r0, NPT)])
            pltpu.sync_copy(qnew, acc.at[pl.ds(r0, NPT)])
            plsc.subcore_barrier()

            def ej(p, icarry):
                j = p * 2
                g0 = pltpu.async_copy(qout.at[srcbuf.at[j]], rw0, sga)
                g1 = pltpu.async_copy(qout.at[srcbuf.at[j + 1]], rw1, sgb)
                g0.wait()
                s0 = pltpu.async_copy(rw0, acc.at[dstbuf.at[j]], ssc,
                                      add=True)
                g1.wait()
                s1 = pltpu.async_copy(rw1, acc.at[dstbuf.at[j + 1]], ssc,
                                      add=True)
                s0.wait()
                s1.wait()
                return icarry

            lax.fori_loop(0, EPT_ROWS // 2, ej, 0)
            plsc.subcore_barrier()
            pltpu.sync_copy(acc.at[pl.ds(r0, NPT)], qnew)

            def cblk(bb, icarry):
                b0 = bb * 128
                pltpu.sync_copy(q0ref.at[pl.ds(r0 + b0, 128)], rw2)

                def cb(i, iicarry):
                    r = b0 + i
                    qnew[r, pl.ds(0, 16)] = (cbuf[r, pl.ds(0, 16)]
                                             * qnew[r, pl.ds(0, 16)]
                                             + rw2[i, pl.ds(0, 16)] * 0.1)
                    qnew[r, pl.ds(16, 16)] = (cbuf[r, pl.ds(16, 16)]
                                              * qnew[r, pl.ds(16, 16)]
                                              + rw2[i, pl.ds(16, 16)] * 0.1)
                    return iicarry

                lax.fori_loop(0, 128, cb, 0)
                return icarry

            lax.fori_loop(0, NPT // 128, cblk, 0)
            return carry

        lax.fori_loop(0, KSTEPS, kbody, 0)
        pltpu.sync_copy(qnew, qout.at[pl.ds(r0, NPT)])

    @pl.when(c == 0)
    def _():
        run(qlo, q0l)

    @pl.when(c == 1)
    def _():
        run(qro, q0r)


def _ls_body(q_ref, sd_ref, o_ref):
    sd = sd_ref[...]
    h = q_ref[...] * jnp.concatenate([sd, sd], axis=1)
    m = jnp.max(h, axis=1, keepdims=True)
    e = jnp.exp(h - m)
    o_ref[...] = (h - m) - jnp.log(jnp.sum(e, axis=1, keepdims=True))


def _run_log_softmax(q, sdeg):
    blk = 1000
    return pl.pallas_call(
        _ls_body,
        grid=(N // blk,),
        in_specs=[
            pl.BlockSpec((blk, NCLS), lambda i: (i, 0)),
            pl.BlockSpec((blk, FH), lambda i: (i, 0)),
        ],
        out_specs=pl.BlockSpec((blk, NCLS), lambda i: (i, 0)),
        out_shape=jax.ShapeDtypeStruct((N, NCLS), jnp.float32),
    )(q, sdeg)


def kernel(x, edge_index, W1, b1, W2, b2):
    src = edge_index[0].astype(jnp.int32)
    dst = edge_index[1].astype(jnp.int32)
    npad = EPAD - E
    src_p = jnp.concatenate([src, jnp.zeros((npad,), jnp.int32)]).reshape(EROWS, ECW)
    dst_flat = jnp.concatenate([dst, jnp.full((npad,), TRASH, jnp.int32)])
    dst_p = dst_flat.reshape(EROWS, ECW)
    dst_p128 = dst_flat.reshape(EPAD // 128, 128)
    zeros_a = jnp.zeros((NACC, FH), jnp.float32)
    ones_a = jnp.ones((128, FH), jnp.float32)

    degl, degr = _deg_kernel(dst_p128, zeros_a, ones_a)
    c9, q0l, q0r, sdeg = _run_mlp(x, W1, b1, W2, b2,
                                  degl[:N], degr[:N])
    pad = ((0, NACC - N), (0, 0))
    ql, qr = _prop_kernel(src_p, dst_p, jnp.pad(c9, pad),
                          jnp.pad(q0l, pad), jnp.pad(q0r, pad))
    q = jnp.concatenate([ql[:N], qr[:N]], axis=1)
    return _run_log_softmax(q, sdeg)
